# baseline (speedup 1.0000x reference)
import functools

import jax
import jax.numpy as jnp
from jax import lax
from jax.experimental import pallas as pl
from jax.experimental.pallas import tpu as pltpu
from jax.experimental.pallas import tpu_sc as plsc

N = 10000
D = 2
I = 32
T = 128
P = 100000
ND = N * D
NB = 2500
NC = 2
NS = 16
NW = NC * NS
TPW = T // NW
CP = 8192
NCH = 13


def _tc_body(times_ref, vr_ref, x0r_ref, u_ref):
    t = times_ref[:]
    j = lax.broadcasted_iota(jnp.int32, (T, I), 1).astype(jnp.float32)
    s = jnp.float32(1.0 / I)
    w = jnp.clip(t - j * s, 0.0, s)
    u = lax.dot_general(w, vr_ref[:], (((1,), (0,)), ((), ())),
                        precision=lax.Precision.HIGHEST,
                        preferred_element_type=jnp.float32)
    u_ref[:] = u + x0r_ref[:]


def _sc_pair_kernel(u, pa, pb, bsq16):
    mesh = plsc.VectorSubcoreMesh(core_axis_name="c", subcore_axis_name="s")

    @functools.partial(
        pl.kernel,
        mesh=mesh,
        compiler_params=pltpu.CompilerParams(
            use_tc_tiling_on_sc=False, needs_layout_passes=False),
        out_type=jax.ShapeDtypeStruct((T, P), jnp.float32),
        scratch_types=[
            pltpu.VMEM((TPW * ND,), jnp.float32),
            pltpu.VMEM((CP,), jnp.int32),
            pltpu.VMEM((CP,), jnp.int32),
            pltpu.VMEM((TPW, CP), jnp.float32),
            pltpu.VMEM((16,), jnp.float32),
        ],
    )
    def body(u_hbm, pa_hbm, pb_hbm, bsq_hbm, out_hbm, tab, ia, ib, obuf, bv):
        wid = lax.axis_index("s") * NC + lax.axis_index("c")
        t0 = wid * TPW
        for k in range(TPW):
            pltpu.sync_copy(u_hbm.at[t0 + k], tab.at[pl.ds(k * ND, ND)])
        pltpu.sync_copy(bsq_hbm, bv)
        bs = bv[...]

        def chunk_step(c, carry):
            base = jnp.minimum(c * CP, P - CP)
            pltpu.sync_copy(pa_hbm.at[pl.ds(base, CP)], ia)
            pltpu.sync_copy(pb_hbm.at[pl.ds(base, CP)], ib)

            @plsc.parallel_loop(0, CP // 16, unroll=4)
            def group_step(g):
                sl = pl.ds(g * 16, 16)
                a2 = ia[sl] * 2
                b2 = ib[sl] * 2
                for k in range(TPW):
                    off = k * ND
                    ua0 = plsc.load_gather(tab, [a2 + off])
                    ua1 = plsc.load_gather(tab, [a2 + (off + 1)])
                    ub0 = plsc.load_gather(tab, [b2 + off])
                    ub1 = plsc.load_gather(tab, [b2 + (off + 1)])
                    d0 = ua0 - ub0
                    d1 = ua1 - ub1
                    obuf[k, sl] = jnp.exp(bs - (d0 * d0 + d1 * d1))

            for k in range(TPW):
                pltpu.sync_copy(obuf.at[k], out_hbm.at[t0 + k, pl.ds(base, CP)])
            return carry

        lax.fori_loop(0, NCH, chunk_step, 0)

    return body(u, pa, pb, bsq16)


def kernel(x0, v, beta, times_list, node_pairs):
    u = pl.pallas_call(
        _tc_body,
        out_shape=jax.ShapeDtypeStruct((T, ND), jnp.float32),
    )(times_list.reshape(T, 1), v.reshape(I, ND), x0.reshape(1, ND))

    bsq16 = jnp.full((16,), beta[0] * beta[0], jnp.float32)
    return _sc_pair_kernel(u, node_pairs[0], node_pairs[1], bsq16)


# speedup vs baseline: 1.0005x; 1.0005x over previous
import functools

import jax
import jax.numpy as jnp
from jax import lax
from jax.experimental import pallas as pl
from jax.experimental.pallas import tpu as pltpu
from jax.experimental.pallas import tpu_sc as plsc

N = 10000
D = 2
I = 32
T = 128
P = 100000
ND = N * D
NB = 2500
NC = 2
NS = 16
NW = NC * NS
TPW = T // NW
CP = 8192
NCH = 13


def _tc_body(times_ref, vr_ref, x0r_ref, u_ref):
    t = times_ref[:]
    j = lax.broadcasted_iota(jnp.int32, (T, I), 1).astype(jnp.float32)
    s = jnp.float32(1.0 / I)
    w = jnp.clip(t - j * s, 0.0, s)
    u = lax.dot_general(w, vr_ref[:], (((1,), (0,)), ((), ())),
                        precision=lax.Precision.HIGHEST,
                        preferred_element_type=jnp.float32)
    u_ref[:] = u + x0r_ref[:]


def _sc_pair_kernel(u, pa, pb, bsq16):
    mesh = plsc.VectorSubcoreMesh(core_axis_name="c", subcore_axis_name="s")

    @functools.partial(
        pl.kernel,
        mesh=mesh,
        compiler_params=pltpu.CompilerParams(
            use_tc_tiling_on_sc=False, needs_layout_passes=False),
        out_type=jax.ShapeDtypeStruct((T, P), jnp.float32),
        scratch_types=[
            pltpu.VMEM((TPW * ND,), jnp.float32),
            pltpu.VMEM((CP,), jnp.int32),
            pltpu.VMEM((CP,), jnp.int32),
            pltpu.VMEM((TPW, CP), jnp.float32),
            pltpu.VMEM((16,), jnp.float32),
        ],
    )
    def body(u_hbm, pa_hbm, pb_hbm, bsq_hbm, out_hbm, tab, ia, ib, obuf, bv):
        wid = lax.axis_index("s") * NC + lax.axis_index("c")
        t0 = wid * TPW
        for k in range(TPW):
            pltpu.sync_copy(u_hbm.at[pl.ds((t0 + k) * ND, ND)],
                            tab.at[pl.ds(k * ND, ND)])
        pltpu.sync_copy(bsq_hbm, bv)
        bs = bv[...]

        def chunk_step(c, carry):
            base = jnp.minimum(c * CP, P - CP)
            pltpu.sync_copy(pa_hbm.at[pl.ds(base, CP)], ia)
            pltpu.sync_copy(pb_hbm.at[pl.ds(base, CP)], ib)

            @plsc.parallel_loop(0, CP // 16, unroll=4)
            def group_step(g):
                sl = pl.ds(g * 16, 16)
                a2 = ia[sl] * 2
                b2 = ib[sl] * 2
                for k in range(TPW):
                    off = k * ND
                    ua0 = plsc.load_gather(tab, [a2 + off])
                    ua1 = plsc.load_gather(tab, [a2 + (off + 1)])
                    ub0 = plsc.load_gather(tab, [b2 + off])
                    ub1 = plsc.load_gather(tab, [b2 + (off + 1)])
                    d0 = ua0 - ub0
                    d1 = ua1 - ub1
                    obuf[k, sl] = jnp.exp(bs - (d0 * d0 + d1 * d1))

            for k in range(TPW):
                pltpu.sync_copy(obuf.at[k], out_hbm.at[t0 + k, pl.ds(base, CP)])
            return carry

        lax.fori_loop(0, NCH, chunk_step, 0)

    return body(u, pa, pb, bsq16)


def kernel(x0, v, beta, times_list, node_pairs):
    u = pl.pallas_call(
        _tc_body,
        out_shape=jax.ShapeDtypeStruct((T, ND), jnp.float32),
    )(times_list.reshape(T, 1), v.reshape(I, ND), x0.reshape(1, ND))

    bsq16 = jnp.full((16,), beta[0] * beta[0], jnp.float32)
    return _sc_pair_kernel(u.reshape(T * ND), node_pairs[0], node_pairs[1], bsq16)


# speedup vs baseline: 1.5046x; 1.5038x over previous
import functools

import jax
import jax.numpy as jnp
from jax import lax
from jax.experimental import pallas as pl
from jax.experimental.pallas import tpu as pltpu
from jax.experimental.pallas import tpu_sc as plsc

N = 10000
D = 2
I = 32
T = 128
P = 100000
DN = D * N
NC = 2
NS = 16
NW = NC * NS
TPW = T // NW
CP = 4096
NCH = 25
TAILB = P - CP


def _tc_body(times_ref, vr_ref, x0r_ref, u_ref):
    t = times_ref[:]
    j = lax.broadcasted_iota(jnp.int32, (T, I), 1).astype(jnp.float32)
    s = jnp.float32(1.0 / I)
    w = jnp.clip(t - j * s, 0.0, s)
    u = lax.dot_general(w, vr_ref[:], (((1,), (0,)), ((), ())),
                        precision=lax.Precision.HIGHEST,
                        preferred_element_type=jnp.float32)
    u_ref[:] = u + x0r_ref[:]


def _sc_pair_kernel(u, pa, pb, bsq16):
    mesh = plsc.VectorSubcoreMesh(core_axis_name="c", subcore_axis_name="s")

    @functools.partial(
        pl.kernel,
        mesh=mesh,
        compiler_params=pltpu.CompilerParams(
            use_tc_tiling_on_sc=False, needs_layout_passes=False),
        out_type=jax.ShapeDtypeStruct((T, P), jnp.float32),
        scratch_types=[
            pltpu.VMEM((TPW * D * N,), jnp.float32),
            pltpu.VMEM((CP,), jnp.int32),
            pltpu.VMEM((CP,), jnp.int32),
            pltpu.VMEM((CP,), jnp.int32),
            pltpu.VMEM((CP,), jnp.int32),
            pltpu.VMEM((TPW, CP), jnp.float32),
            pltpu.VMEM((TPW, CP), jnp.float32),
            pltpu.VMEM((16,), jnp.float32),
            pltpu.SemaphoreType.DMA,
            pltpu.SemaphoreType.DMA,
            pltpu.SemaphoreType.DMA,
            pltpu.SemaphoreType.DMA,
        ],
    )
    def body(u_hbm, pa_hbm, pb_hbm, bsq_hbm, out_hbm,
             tab, ia0, ia1, ib0, ib1, ob0, ob1, bv,
             psem0, psem1, osem0, osem1):
        wid = lax.axis_index("s") * NC + lax.axis_index("c")
        t0 = wid * TPW
        for k in range(TPW):
            for d in range(D):
                pltpu.sync_copy(u_hbm.at[(t0 + k) * D + d],
                                tab.at[pl.ds((k * D + d) * N, N)])
        pltpu.sync_copy(bsq_hbm, bv)
        bs = bv[...]

        def compute_store(base, ia, ib, ob, osem):
            @plsc.parallel_loop(0, CP // 16, unroll=4)
            def group_step(g):
                sl = pl.ds(g * 16, 16)
                av = ia[sl]
                bv_ = ib[sl]
                for k in range(TPW):
                    r0 = tab.at[pl.ds((k * D) * N, N)]
                    r1 = tab.at[pl.ds((k * D + 1) * N, N)]
                    d0 = plsc.load_gather(r0, [av]) - plsc.load_gather(r0, [bv_])
                    d1 = plsc.load_gather(r1, [av]) - plsc.load_gather(r1, [bv_])
                    ob[k, sl] = jnp.exp(bs - (d0 * d0 + d1 * d1))

            for k in range(TPW):
                pltpu.async_copy(ob.at[k], out_hbm.at[t0 + k, pl.ds(base, CP)],
                                 osem)

        def wait_pair(ia, ib, psem):
            pltpu.make_async_copy(pa_hbm.at[pl.ds(0, CP)], ia, psem).wait()
            pltpu.make_async_copy(pb_hbm.at[pl.ds(0, CP)], ib, psem).wait()

        def fetch_pair(base, ia, ib, psem):
            pltpu.async_copy(pa_hbm.at[pl.ds(base, CP)], ia, psem)
            pltpu.async_copy(pb_hbm.at[pl.ds(base, CP)], ib, psem)

        def drain_out(ob, osem):
            for k in range(TPW):
                pltpu.make_async_copy(ob.at[k], out_hbm.at[0, pl.ds(0, CP)],
                                      osem).wait()

        fetch_pair(0, ia0, ib0, psem0)

        def chunk2_step(j, carry):
            c0 = 2 * j
            wait_pair(ia0, ib0, psem0)
            fetch_pair((c0 + 1) * CP, ia1, ib1, psem1)

            @pl.when(j >= 1)
            def _():
                drain_out(ob0, osem0)

            compute_store(c0 * CP, ia0, ib0, ob0, osem0)

            wait_pair(ia1, ib1, psem1)
            fetch_pair(jnp.minimum((c0 + 2) * CP, TAILB), ia0, ib0, psem0)

            @pl.when(j >= 1)
            def _():
                drain_out(ob1, osem1)

            compute_store((c0 + 1) * CP, ia1, ib1, ob1, osem1)
            return carry

        lax.fori_loop(0, (NCH - 1) // 2, chunk2_step, 0)

        wait_pair(ia0, ib0, psem0)
        drain_out(ob0, osem0)
        compute_store(TAILB, ia0, ib0, ob0, osem0)
        drain_out(ob0, osem0)
        drain_out(ob1, osem1)

    return body(u, pa, pb, bsq16)


def kernel(x0, v, beta, times_list, node_pairs):
    vr = jnp.transpose(v, (0, 2, 1)).reshape(I, DN)
    x0r = jnp.transpose(x0, (1, 0)).reshape(1, DN)
    u = pl.pallas_call(
        _tc_body,
        out_shape=jax.ShapeDtypeStruct((T, DN), jnp.float32),
    )(times_list.reshape(T, 1), vr, x0r)

    bsq16 = jnp.full((16,), beta[0] * beta[0], jnp.float32)
    return _sc_pair_kernel(u.reshape(T * D, N), node_pairs[0], node_pairs[1],
                           bsq16)


# speedup vs baseline: 1.5059x; 1.0008x over previous
import functools

import jax
import jax.numpy as jnp
from jax import lax
from jax.experimental import pallas as pl
from jax.experimental.pallas import tpu as pltpu
from jax.experimental.pallas import tpu_sc as plsc

N = 10000
D = 2
I = 32
T = 128
P = 100000
DN = D * N
NC = 2
NS = 16
NW = NC * NS
TPW = T // NW
CP = 4096
NCH = 25
TAILB = P - CP


def _tc_body(times_ref, vr_ref, x0r_ref, u_ref):
    t = times_ref[:]
    j = lax.broadcasted_iota(jnp.int32, (T, I), 1).astype(jnp.float32)
    s = jnp.float32(1.0 / I)
    w = jnp.clip(t - j * s, 0.0, s)
    u = lax.dot_general(w, vr_ref[:], (((1,), (0,)), ((), ())),
                        precision=lax.Precision.HIGHEST,
                        preferred_element_type=jnp.float32)
    u_ref[:] = u + x0r_ref[:]


def _sc_pair_kernel(u, pa, pb, bsq16):
    mesh = plsc.VectorSubcoreMesh(core_axis_name="c", subcore_axis_name="s")

    @functools.partial(
        pl.kernel,
        mesh=mesh,
        compiler_params=pltpu.CompilerParams(
            use_tc_tiling_on_sc=False, needs_layout_passes=False),
        out_type=jax.ShapeDtypeStruct((T, P), jnp.float32),
        scratch_types=[
            pltpu.VMEM((TPW * D * N,), jnp.float32),
            pltpu.VMEM((CP,), jnp.int32),
            pltpu.VMEM((CP,), jnp.int32),
            pltpu.VMEM((CP,), jnp.int32),
            pltpu.VMEM((CP,), jnp.int32),
            pltpu.VMEM((TPW, CP), jnp.float32),
            pltpu.VMEM((TPW, CP), jnp.float32),
            pltpu.VMEM((16,), jnp.float32),
            pltpu.SemaphoreType.DMA,
            pltpu.SemaphoreType.DMA,
            pltpu.SemaphoreType.DMA,
            pltpu.SemaphoreType.DMA,
        ],
    )
    def body(u_hbm, pa_hbm, pb_hbm, bsq_hbm, out_hbm,
             tab, ia0, ia1, ib0, ib1, ob0, ob1, bv,
             psem0, psem1, osem0, osem1):
        wid = lax.axis_index("s") * NC + lax.axis_index("c")
        t0 = wid * TPW
        for k in range(TPW):
            for d in range(D):
                pltpu.sync_copy(u_hbm.at[(t0 + k) * D + d],
                                tab.at[pl.ds((k * D + d) * N, N)])
        pltpu.sync_copy(bsq_hbm, bv)
        bs = bv[...]

        def compute_store(base, ia, ib, ob, osem):
            @plsc.parallel_loop(0, CP // 16, unroll=8)
            def group_step(g):
                sl = pl.ds(g * 16, 16)
                av = ia[sl]
                bv_ = ib[sl]
                for k in range(TPW):
                    r0 = tab.at[pl.ds((k * D) * N, N)]
                    r1 = tab.at[pl.ds((k * D + 1) * N, N)]
                    d0 = plsc.load_gather(r0, [av]) - plsc.load_gather(r0, [bv_])
                    d1 = plsc.load_gather(r1, [av]) - plsc.load_gather(r1, [bv_])
                    ob[k, sl] = jnp.exp(bs - (d0 * d0 + d1 * d1))

            for k in range(TPW):
                pltpu.async_copy(ob.at[k], out_hbm.at[t0 + k, pl.ds(base, CP)],
                                 osem)

        def wait_pair(ia, ib, psem):
            pltpu.make_async_copy(pa_hbm.at[pl.ds(0, CP)], ia, psem).wait()
            pltpu.make_async_copy(pb_hbm.at[pl.ds(0, CP)], ib, psem).wait()

        def fetch_pair(base, ia, ib, psem):
            pltpu.async_copy(pa_hbm.at[pl.ds(base, CP)], ia, psem)
            pltpu.async_copy(pb_hbm.at[pl.ds(base, CP)], ib, psem)

        def drain_out(ob, osem):
            for k in range(TPW):
                pltpu.make_async_copy(ob.at[k], out_hbm.at[0, pl.ds(0, CP)],
                                      osem).wait()

        fetch_pair(0, ia0, ib0, psem0)

        def chunk2_step(j, carry):
            c0 = 2 * j
            wait_pair(ia0, ib0, psem0)
            fetch_pair((c0 + 1) * CP, ia1, ib1, psem1)

            @pl.when(j >= 1)
            def _():
                drain_out(ob0, osem0)

            compute_store(c0 * CP, ia0, ib0, ob0, osem0)

            wait_pair(ia1, ib1, psem1)
            fetch_pair(jnp.minimum((c0 + 2) * CP, TAILB), ia0, ib0, psem0)

            @pl.when(j >= 1)
            def _():
                drain_out(ob1, osem1)

            compute_store((c0 + 1) * CP, ia1, ib1, ob1, osem1)
            return carry

        lax.fori_loop(0, (NCH - 1) // 2, chunk2_step, 0)

        wait_pair(ia0, ib0, psem0)
        drain_out(ob0, osem0)
        compute_store(TAILB, ia0, ib0, ob0, osem0)
        drain_out(ob0, osem0)
        drain_out(ob1, osem1)

    return body(u, pa, pb, bsq16)


def kernel(x0, v, beta, times_list, node_pairs):
    vr = jnp.transpose(v, (0, 2, 1)).reshape(I, DN)
    x0r = jnp.transpose(x0, (1, 0)).reshape(1, DN)
    u = pl.pallas_call(
        _tc_body,
        out_shape=jax.ShapeDtypeStruct((T, DN), jnp.float32),
    )(times_list.reshape(T, 1), vr, x0r)

    bsq16 = jnp.full((16,), beta[0] * beta[0], jnp.float32)
    return _sc_pair_kernel(u.reshape(T * D, N), node_pairs[0], node_pairs[1],
                           bsq16)


# speedup vs baseline: 1.5685x; 1.0416x over previous
import functools

import jax
import jax.numpy as jnp
from jax import lax
from jax.experimental import pallas as pl
from jax.experimental.pallas import tpu as pltpu
from jax.experimental.pallas import tpu_sc as plsc

N = 10000
D = 2
I = 32
T = 128
P = 100000
DN = D * N
DNP = 20096
NC = 2
NS = 16
NW = NC * NS
TPW = T // NW
CP = 4096
NCH = 25
TAILB = P - CP


def _tc_body(times_ref, vr_ref, x0r_ref, u_ref):
    t = times_ref[:]
    j = lax.broadcasted_iota(jnp.int32, (T, I), 1).astype(jnp.float32)
    s = jnp.float32(1.0 / I)
    w = jnp.clip(t - j * s, 0.0, s)
    u = lax.dot_general(w, vr_ref[:], (((1,), (0,)), ((), ())),
                        precision=lax.Precision.HIGHEST,
                        preferred_element_type=jnp.float32)
    u = u + x0r_ref[:]
    for t_ in range(T):
        u_ref[pl.ds(t_ * DNP, DNP)] = u[t_]


def _sc_pair_kernel(u, pa, pb, bsq16):
    mesh = plsc.VectorSubcoreMesh(core_axis_name="c", subcore_axis_name="s")

    @functools.partial(
        pl.kernel,
        mesh=mesh,
        compiler_params=pltpu.CompilerParams(
            use_tc_tiling_on_sc=False, needs_layout_passes=False),
        out_type=jax.ShapeDtypeStruct((T, P), jnp.float32),
        scratch_types=[
            pltpu.VMEM((TPW * D * N,), jnp.float32),
            pltpu.VMEM((CP,), jnp.int32),
            pltpu.VMEM((CP,), jnp.int32),
            pltpu.VMEM((CP,), jnp.int32),
            pltpu.VMEM((CP,), jnp.int32),
            pltpu.VMEM((TPW, CP), jnp.float32),
            pltpu.VMEM((TPW, CP), jnp.float32),
            pltpu.VMEM((16,), jnp.float32),
            pltpu.SemaphoreType.DMA,
            pltpu.SemaphoreType.DMA,
            pltpu.SemaphoreType.DMA,
            pltpu.SemaphoreType.DMA,
        ],
    )
    def body(u_hbm, pa_hbm, pb_hbm, bsq_hbm, out_hbm,
             tab, ia0, ia1, ib0, ib1, ob0, ob1, bv,
             psem0, psem1, osem0, osem1):
        wid = lax.axis_index("s") * NC + lax.axis_index("c")
        t0 = wid * TPW
        for k in range(TPW):
            for d in range(D):
                pltpu.sync_copy(u_hbm.at[pl.ds((t0 + k) * DNP + d * N, N)],
                                tab.at[pl.ds((k * D + d) * N, N)])
        pltpu.sync_copy(bsq_hbm, bv)
        bs = bv[...]

        def compute_store(base, ia, ib, ob, osem):
            @plsc.parallel_loop(0, CP // 16, unroll=8)
            def group_step(g):
                sl = pl.ds(g * 16, 16)
                av = ia[sl]
                bv_ = ib[sl]
                for k in range(TPW):
                    r0 = tab.at[pl.ds((k * D) * N, N)]
                    r1 = tab.at[pl.ds((k * D + 1) * N, N)]
                    d0 = plsc.load_gather(r0, [av]) - plsc.load_gather(r0, [bv_])
                    d1 = plsc.load_gather(r1, [av]) - plsc.load_gather(r1, [bv_])
                    ob[k, sl] = jnp.exp(bs - (d0 * d0 + d1 * d1))

            for k in range(TPW):
                pltpu.async_copy(ob.at[k], out_hbm.at[t0 + k, pl.ds(base, CP)],
                                 osem)

        def wait_pair(ia, ib, psem):
            pltpu.make_async_copy(pa_hbm.at[pl.ds(0, CP)], ia, psem).wait()
            pltpu.make_async_copy(pb_hbm.at[pl.ds(0, CP)], ib, psem).wait()

        def fetch_pair(base, ia, ib, psem):
            pltpu.async_copy(pa_hbm.at[pl.ds(base, CP)], ia, psem)
            pltpu.async_copy(pb_hbm.at[pl.ds(base, CP)], ib, psem)

        def drain_out(ob, osem):
            for k in range(TPW):
                pltpu.make_async_copy(ob.at[k], out_hbm.at[0, pl.ds(0, CP)],
                                      osem).wait()

        fetch_pair(0, ia0, ib0, psem0)

        def chunk2_step(j, carry):
            c0 = 2 * j
            wait_pair(ia0, ib0, psem0)
            fetch_pair((c0 + 1) * CP, ia1, ib1, psem1)

            @pl.when(j >= 1)
            def _():
                drain_out(ob0, osem0)

            compute_store(c0 * CP, ia0, ib0, ob0, osem0)

            wait_pair(ia1, ib1, psem1)
            fetch_pair(jnp.minimum((c0 + 2) * CP, TAILB), ia0, ib0, psem0)

            @pl.when(j >= 1)
            def _():
                drain_out(ob1, osem1)

            compute_store((c0 + 1) * CP, ia1, ib1, ob1, osem1)
            return carry

        lax.fori_loop(0, (NCH - 1) // 2, chunk2_step, 0)

        wait_pair(ia0, ib0, psem0)
        drain_out(ob0, osem0)
        compute_store(TAILB, ia0, ib0, ob0, osem0)
        drain_out(ob0, osem0)
        drain_out(ob1, osem1)

    return body(u, pa, pb, bsq16)


def kernel(x0, v, beta, times_list, node_pairs):
    pad = DNP - DN
    vr = jnp.pad(jnp.transpose(v, (0, 2, 1)).reshape(I, DN), ((0, 0), (0, pad)))
    x0r = jnp.pad(jnp.transpose(x0, (1, 0)).reshape(1, DN), ((0, 0), (0, pad)))
    u = pl.pallas_call(
        _tc_body,
        out_shape=jax.ShapeDtypeStruct((T * DNP,), jnp.float32),
    )(times_list.reshape(T, 1), vr, x0r)

    bsq16 = jnp.full((16,), beta[0] * beta[0], jnp.float32)
    return _sc_pair_kernel(u, node_pairs[0], node_pairs[1], bsq16)
